# baseline (speedup 1.0000x reference)
import functools

import jax
import jax.numpy as jnp
from jax import lax
from jax.experimental import pallas as pl
from jax.experimental.pallas import tpu as pltpu
from jax.experimental.pallas import tpu_sc as plsc

_B = 16384
_D = 64
_K = 20
_VOC = 1000000
_NC = 2
_NS = 16
_NW = _NC * _NS
_CH = 128

_CEN_CH = _B // _NW // _CH
_NEG_ROWS = _B * _K
_NEG_CH = _NEG_ROWS // _NW // _CH
_PD = 2 * _D


def _sc_gather_body(cen_idx, ctx_idx, neg_idx, w_cen, w_ctx,
                    out_cen, out_ctx, out_neg,
                    idx_cen_v, idx_ctx_v, idx_neg_v, bufs, gsem, ssem):
    wid = lax.axis_index("s") * _NC + lax.axis_index("c")
    pltpu.sync_copy(cen_idx.at[wid], idx_cen_v)
    pltpu.sync_copy(ctx_idx.at[wid], idx_ctx_v)
    pltpu.sync_copy(neg_idx.at[wid], idx_neg_v)

    def run(table, idx_v, nch, out_hbm):
        base = wid * nch * _CH

        def g_start(j, s):
            pltpu.make_async_copy(table.at[idx_v.at[j]], bufs.at[s],
                                  gsem.at[s]).start()

        def g_wait(j, s):
            pltpu.make_async_copy(table.at[idx_v.at[j]], bufs.at[s],
                                  gsem.at[s]).wait()

        def s_start(j, s):
            pltpu.make_async_copy(bufs.at[s],
                                  out_hbm.at[pl.ds(base + j * _CH, _CH)],
                                  ssem.at[s]).start()

        def s_wait(j, s):
            pltpu.make_async_copy(bufs.at[s],
                                  out_hbm.at[pl.ds(base + j * _CH, _CH)],
                                  ssem.at[s]).wait()

        nhalf = nch // 2
        g_start(0, 0)
        g_start(1, 1)

        def body(j2, carry):
            j = 2 * j2
            cur = lax.rem(j2, 2) * 2
            nxt = 2 - cur
            g_wait(j, cur)
            g_wait(j + 1, cur + 1)
            s_start(j, cur)
            s_start(j + 1, cur + 1)

            @pl.when(j2 >= 1)
            def _():
                s_wait(j - 2, nxt)
                s_wait(j - 1, nxt + 1)

            @pl.when(j2 + 1 < nhalf)
            def _():
                g_start(j + 2, nxt)
                g_start(j + 3, nxt + 1)

            return carry

        lax.fori_loop(0, nhalf, body, 0)
        last = (nhalf - 1) % 2 * 2
        s_wait(nch - 2, last)
        s_wait(nch - 1, last + 1)

    run(w_cen, idx_cen_v, _CEN_CH, out_cen)
    run(w_ctx, idx_ctx_v, _CEN_CH, out_ctx)
    run(w_ctx, idx_neg_v, _NEG_CH, out_neg)


_sc_gather = pl.kernel(
    _sc_gather_body,
    out_type=(
        jax.ShapeDtypeStruct((_B, _PD), jnp.float32),
        jax.ShapeDtypeStruct((_B, _PD), jnp.float32),
        jax.ShapeDtypeStruct((_NEG_ROWS, _PD), jnp.float32),
    ),
    mesh=plsc.VectorSubcoreMesh(core_axis_name="c", subcore_axis_name="s",
                                num_cores=_NC, num_subcores=_NS),
    scratch_types=[
        pltpu.VMEM((_CEN_CH, _CH), jnp.int32),
        pltpu.VMEM((_CEN_CH, _CH), jnp.int32),
        pltpu.VMEM((_NEG_CH, _CH), jnp.int32),
        pltpu.VMEM((4, _CH, _PD), jnp.float32),
        pltpu.SemaphoreType.DMA((4,)),
        pltpu.SemaphoreType.DMA((4,)),
    ],
)


_BLK = 1024
_NBLK = _B // _BLK


def _log_sigmoid(x):
    return jnp.minimum(x, 0.0) - jnp.log1p(jnp.exp(-jnp.abs(x)))


def _tc_body(cen_ref, ctx_ref, neg_ref, hcen_ref, hctx_ref, hneg_ref,
             out_ref, c_acc, s_acc):
    i = pl.program_id(0)

    @pl.when(i == 0)
    def _():
        c_acc[...] = jnp.zeros_like(c_acc)
        s_acc[0, 0] = 0.0

    cen2 = cen_ref[...]
    ctx2 = ctx_ref[...]
    neg2 = neg_ref[...]
    cen = jnp.where(hcen_ref[...] == 1, cen2[:, _D:], cen2[:, :_D])
    ctx = jnp.where(hctx_ref[...] == 1, ctx2[:, _D:], ctx2[:, :_D])
    neg = jnp.where(hneg_ref[...][:, :, None] == 1,
                    neg2[:, :, _D:], neg2[:, :, :_D])
    c_acc[...] += lax.dot_general(ctx, cen, (((0,), (0,)), ((), ())),
                                  preferred_element_type=jnp.float32)
    rest = jnp.sum(neg * cen[:, None, :], axis=-1)
    s_acc[0, 0] += jnp.sum(_log_sigmoid(-rest))

    @pl.when(i == _NBLK - 1)
    def _():
        out_ref[0, 0] = -(jnp.mean(_log_sigmoid(c_acc[...])) + s_acc[0, 0])


_tc_reduce = pl.pallas_call(
    _tc_body,
    grid=(_NBLK,),
    in_specs=[
        pl.BlockSpec((_BLK, _PD), lambda i: (i, 0)),
        pl.BlockSpec((_BLK, _PD), lambda i: (i, 0)),
        pl.BlockSpec((_BLK, _K, _PD), lambda i: (i, 0, 0)),
        pl.BlockSpec((_BLK, 1), lambda i: (i, 0)),
        pl.BlockSpec((_BLK, 1), lambda i: (i, 0)),
        pl.BlockSpec((_BLK, _K), lambda i: (i, 0)),
    ],
    out_specs=pl.BlockSpec(memory_space=pltpu.SMEM),
    out_shape=jax.ShapeDtypeStruct((1, 1), jnp.float32),
    scratch_shapes=[
        pltpu.VMEM((_D, _D), jnp.float32),
        pltpu.SMEM((1, 1), jnp.float32),
    ],
)


def kernel(central, context, neg_samples, W_central, W_context):
    cen = central.astype(jnp.int32)
    ctx = context.astype(jnp.int32)
    neg = neg_samples.astype(jnp.int32)
    cen_g = (cen >> 1).reshape(_NW, _CEN_CH, _CH)
    ctx_g = (ctx >> 1).reshape(_NW, _CEN_CH, _CH)
    neg_g = (neg >> 1).reshape(_NW, _NEG_CH, _CH)
    wp_cen = W_central.reshape(_VOC // 2, _PD)
    wp_ctx = W_context.reshape(_VOC // 2, _PD)
    emb_cen, emb_ctx, emb_neg = _sc_gather(cen_g, ctx_g, neg_g,
                                           wp_cen, wp_ctx)
    loss = _tc_reduce(emb_cen, emb_ctx, emb_neg.reshape(_B, _K, _PD),
                      (cen & 1).reshape(_B, 1), (ctx & 1).reshape(_B, 1),
                      neg & 1)
    return loss.reshape(())


# speedup vs baseline: 1.4673x; 1.4673x over previous
import functools

import jax
import jax.numpy as jnp
from jax import lax
from jax.experimental import pallas as pl
from jax.experimental.pallas import tpu as pltpu
from jax.experimental.pallas import tpu_sc as plsc

_B = 16384
_D = 64
_K = 20
_VOC = 1000000
_NC = 2
_NS = 16
_NW = _NC * _NS
_CH = 128
_PD = 2 * _D

_CEN_CH = _B // _NW // _CH
_NEG_ROWS = _B * _K
_NEG_CH = _NEG_ROWS // _NW // _CH

_T = 256
_FULL = (_VOC // _T) * _T
_NRANGE = _FULL // _T
_TAIL = _VOC - _FULL


_TB = 4096
_TGRID = 123
_HV = _TB * _TGRID


def _tp_body(lo_ref, hi_ref, out_ref):
    eye = (lax.broadcasted_iota(jnp.int32, (_D, _D), 0)
           == lax.broadcasted_iota(jnp.int32, (_D, _D), 1)
           ).astype(jnp.float32)
    lo_t = lax.dot_general(lo_ref[...], eye, (((0,), (0,)), ((), ())),
                           preferred_element_type=jnp.float32)
    hi_t = lax.dot_general(hi_ref[...], eye, (((0,), (0,)), ((), ())),
                           preferred_element_type=jnp.float32)
    out_ref[...] = jnp.concatenate([lo_t, hi_t], axis=1)


def _tc_transpose(wt):
    return pl.pallas_call(
        _tp_body,
        grid=(_TGRID,),
        in_specs=[
            pl.BlockSpec((_D, _TB), lambda i: (0, i)),
            pl.BlockSpec((_D, _TB),
                         lambda i: (0, jnp.minimum(i + _TGRID,
                                                   _VOC // _TB))),
        ],
        out_specs=pl.BlockSpec((_TB, _PD), lambda i: (i, 0)),
        out_shape=jax.ShapeDtypeStruct((_HV, _PD), jnp.float32),
    )(wt, wt)


def _sc_gather_body(cen_idx, ctx_idx, neg_idx, tcen, tctx,
                    out_cen, out_ctx, out_neg,
                    idx_cen_v, idx_ctx_v, idx_neg_v, bufs, gsem, ssem):
    wid = lax.axis_index("s") * _NC + lax.axis_index("c")
    pltpu.sync_copy(cen_idx.at[wid], idx_cen_v)
    pltpu.sync_copy(ctx_idx.at[wid], idx_ctx_v)
    pltpu.sync_copy(neg_idx.at[wid], idx_neg_v)

    def run(table, idx_v, nch, out_hbm):
        base = wid * nch * _CH

        def g_start(j, s):
            pltpu.make_async_copy(table.at[idx_v.at[j]], bufs.at[s],
                                  gsem.at[s]).start()

        def g_wait(j, s):
            pltpu.make_async_copy(table.at[idx_v.at[j]], bufs.at[s],
                                  gsem.at[s]).wait()

        def s_start(j, s):
            pltpu.make_async_copy(bufs.at[s],
                                  out_hbm.at[pl.ds(base + j * _CH, _CH)],
                                  ssem.at[s]).start()

        def s_wait(j, s):
            pltpu.make_async_copy(bufs.at[s],
                                  out_hbm.at[pl.ds(base + j * _CH, _CH)],
                                  ssem.at[s]).wait()

        nhalf = nch // 2
        g_start(0, 0)
        g_start(1, 1)

        def body(j2, carry):
            j = 2 * j2
            cur = lax.rem(j2, 2) * 2
            nxt = 2 - cur
            g_wait(j, cur)
            g_wait(j + 1, cur + 1)
            s_start(j, cur)
            s_start(j + 1, cur + 1)

            @pl.when(j2 >= 1)
            def _():
                s_wait(j - 2, nxt)
                s_wait(j - 1, nxt + 1)

            @pl.when(j2 + 1 < nhalf)
            def _():
                g_start(j + 2, nxt)
                g_start(j + 3, nxt + 1)

            return carry

        lax.fori_loop(0, nhalf, body, 0)
        last = (nhalf - 1) % 2 * 2
        s_wait(nch - 2, last)
        s_wait(nch - 1, last + 1)

    run(tcen, idx_cen_v, _CEN_CH, out_cen)
    run(tctx, idx_ctx_v, _CEN_CH, out_ctx)
    run(tctx, idx_neg_v, _NEG_CH, out_neg)


_sc_gather = pl.kernel(
    _sc_gather_body,
    out_type=(
        jax.ShapeDtypeStruct((_B, _PD), jnp.float32),
        jax.ShapeDtypeStruct((_B, _PD), jnp.float32),
        jax.ShapeDtypeStruct((_NEG_ROWS, _PD), jnp.float32),
    ),
    mesh=plsc.VectorSubcoreMesh(core_axis_name="c", subcore_axis_name="s",
                                num_cores=_NC, num_subcores=_NS),
    scratch_types=[
        pltpu.VMEM((_CEN_CH, _CH), jnp.int32),
        pltpu.VMEM((_CEN_CH, _CH), jnp.int32),
        pltpu.VMEM((_NEG_CH, _CH), jnp.int32),
        pltpu.VMEM((4, _CH, _PD), jnp.float32),
        pltpu.SemaphoreType.DMA((4,)),
        pltpu.SemaphoreType.DMA((4,)),
    ],
)


_BLK = 1024
_NBLK = _B // _BLK


def _log_sigmoid(x):
    return jnp.minimum(x, 0.0) - jnp.log1p(jnp.exp(-jnp.abs(x)))


def _tc_body(cen_ref, ctx_ref, neg_ref, hcen_ref, hctx_ref, hneg_ref,
             out_ref, c_acc, s_acc):
    i = pl.program_id(0)

    @pl.when(i == 0)
    def _():
        c_acc[...] = jnp.zeros_like(c_acc)
        s_acc[0, 0] = 0.0

    cen2 = cen_ref[...]
    ctx2 = ctx_ref[...]
    neg2 = neg_ref[...]
    cen = jnp.where(hcen_ref[...] == 1, cen2[:, _D:], cen2[:, :_D])
    ctx = jnp.where(hctx_ref[...] == 1, ctx2[:, _D:], ctx2[:, :_D])
    neg = jnp.where(hneg_ref[...][:, :, None] == 1,
                    neg2[:, :, _D:], neg2[:, :, :_D])
    c_acc[...] += lax.dot_general(ctx, cen, (((0,), (0,)), ((), ())),
                                  preferred_element_type=jnp.float32)
    rest = jnp.sum(neg * cen[:, None, :], axis=-1)
    s_acc[0, 0] += jnp.sum(_log_sigmoid(-rest))

    @pl.when(i == _NBLK - 1)
    def _():
        out_ref[0, 0] = -(jnp.mean(_log_sigmoid(c_acc[...])) + s_acc[0, 0])


_tc_reduce = pl.pallas_call(
    _tc_body,
    grid=(_NBLK,),
    in_specs=[
        pl.BlockSpec((_BLK, _PD), lambda i: (i, 0)),
        pl.BlockSpec((_BLK, _PD), lambda i: (i, 0)),
        pl.BlockSpec((_BLK, _K, _PD), lambda i: (i, 0, 0)),
        pl.BlockSpec((_BLK, 1), lambda i: (i, 0)),
        pl.BlockSpec((_BLK, 1), lambda i: (i, 0)),
        pl.BlockSpec((_BLK, _K), lambda i: (i, 0)),
    ],
    out_specs=pl.BlockSpec(memory_space=pltpu.SMEM),
    out_shape=jax.ShapeDtypeStruct((1, 1), jnp.float32),
    scratch_shapes=[
        pltpu.VMEM((_D, _D), jnp.float32),
        pltpu.SMEM((1, 1), jnp.float32),
    ],
)


def kernel(central, context, neg_samples, W_central, W_context):
    cen = central.astype(jnp.int32)
    ctx = context.astype(jnp.int32)
    neg = neg_samples.astype(jnp.int32)

    def prow(v):
        return jnp.where(v >= _HV, v - _HV, v)

    def half(v):
        return (v >= _HV).astype(jnp.int32)

    cen_g = prow(cen).reshape(_NW, _CEN_CH, _CH)
    ctx_g = prow(ctx).reshape(_NW, _CEN_CH, _CH)
    neg_g = prow(neg).reshape(_NW, _NEG_CH, _CH)
    tcen = _tc_transpose(W_central.T)
    tctx = _tc_transpose(W_context.T)
    emb_cen, emb_ctx, emb_neg = _sc_gather(cen_g, ctx_g, neg_g, tcen, tctx)
    loss = _tc_reduce(emb_cen, emb_ctx, emb_neg.reshape(_B, _K, _PD),
                      half(cen).reshape(_B, 1), half(ctx).reshape(_B, 1),
                      half(neg))
    return loss.reshape(())


# speedup vs baseline: 1.4698x; 1.0017x over previous
import functools

import jax
import jax.numpy as jnp
from jax import lax
from jax.experimental import pallas as pl
from jax.experimental.pallas import tpu as pltpu
from jax.experimental.pallas import tpu_sc as plsc

_B = 16384
_D = 64
_K = 20
_VOC = 1000000
_NC = 2
_NS = 16
_NW = _NC * _NS
_CH = 128
_PD = 2 * _D

_CEN_CH = _B // _NW // _CH
_NEG_ROWS = _B * _K
_NEG_CH = _NEG_ROWS // _NW // _CH

_T = 256
_FULL = (_VOC // _T) * _T
_NRANGE = _FULL // _T
_TAIL = _VOC - _FULL


_TB = 4096
_TGRID = 123
_HV = _TB * _TGRID


def _tp_body(lo_ref, hi_ref, out_ref):
    lo_t = jnp.swapaxes(lo_ref[...], 0, 1)
    hi_t = jnp.swapaxes(hi_ref[...], 0, 1)
    out_ref[...] = jnp.concatenate([lo_t, hi_t], axis=1)


def _tc_transpose(wt):
    return pl.pallas_call(
        _tp_body,
        grid=(_TGRID,),
        in_specs=[
            pl.BlockSpec((_D, _TB), lambda i: (0, i)),
            pl.BlockSpec((_D, _TB),
                         lambda i: (0, jnp.minimum(i + _TGRID,
                                                   _VOC // _TB))),
        ],
        out_specs=pl.BlockSpec((_TB, _PD), lambda i: (i, 0)),
        out_shape=jax.ShapeDtypeStruct((_HV, _PD), jnp.float32),
    )(wt, wt)


def _sc_gather_body(cen_idx, ctx_idx, neg_idx, tcen, tctx,
                    out_cen, out_ctx, out_neg,
                    idx_cen_v, idx_ctx_v, idx_neg_v, bufs, gsem, ssem):
    wid = lax.axis_index("s") * _NC + lax.axis_index("c")
    pltpu.sync_copy(cen_idx.at[wid], idx_cen_v)
    pltpu.sync_copy(ctx_idx.at[wid], idx_ctx_v)
    pltpu.sync_copy(neg_idx.at[wid], idx_neg_v)

    def run(table, idx_v, nch, out_hbm):
        base = wid * nch * _CH

        def g_start(j, s):
            pltpu.make_async_copy(table.at[idx_v.at[j]], bufs.at[s],
                                  gsem.at[s]).start()

        def g_wait(j, s):
            pltpu.make_async_copy(table.at[idx_v.at[j]], bufs.at[s],
                                  gsem.at[s]).wait()

        def s_start(j, s):
            pltpu.make_async_copy(bufs.at[s],
                                  out_hbm.at[pl.ds(base + j * _CH, _CH)],
                                  ssem.at[s]).start()

        def s_wait(j, s):
            pltpu.make_async_copy(bufs.at[s],
                                  out_hbm.at[pl.ds(base + j * _CH, _CH)],
                                  ssem.at[s]).wait()

        nhalf = nch // 2
        g_start(0, 0)
        g_start(1, 1)

        def body(j2, carry):
            j = 2 * j2
            cur = lax.rem(j2, 2) * 2
            nxt = 2 - cur
            g_wait(j, cur)
            g_wait(j + 1, cur + 1)
            s_start(j, cur)
            s_start(j + 1, cur + 1)

            @pl.when(j2 >= 1)
            def _():
                s_wait(j - 2, nxt)
                s_wait(j - 1, nxt + 1)

            @pl.when(j2 + 1 < nhalf)
            def _():
                g_start(j + 2, nxt)
                g_start(j + 3, nxt + 1)

            return carry

        lax.fori_loop(0, nhalf, body, 0)
        last = (nhalf - 1) % 2 * 2
        s_wait(nch - 2, last)
        s_wait(nch - 1, last + 1)

    run(tcen, idx_cen_v, _CEN_CH, out_cen)
    run(tctx, idx_ctx_v, _CEN_CH, out_ctx)
    run(tctx, idx_neg_v, _NEG_CH, out_neg)


_sc_gather = pl.kernel(
    _sc_gather_body,
    out_type=(
        jax.ShapeDtypeStruct((_B, _PD), jnp.float32),
        jax.ShapeDtypeStruct((_B, _PD), jnp.float32),
        jax.ShapeDtypeStruct((_NEG_ROWS, _PD), jnp.float32),
    ),
    mesh=plsc.VectorSubcoreMesh(core_axis_name="c", subcore_axis_name="s",
                                num_cores=_NC, num_subcores=_NS),
    scratch_types=[
        pltpu.VMEM((_CEN_CH, _CH), jnp.int32),
        pltpu.VMEM((_CEN_CH, _CH), jnp.int32),
        pltpu.VMEM((_NEG_CH, _CH), jnp.int32),
        pltpu.VMEM((4, _CH, _PD), jnp.float32),
        pltpu.SemaphoreType.DMA((4,)),
        pltpu.SemaphoreType.DMA((4,)),
    ],
)


_BLK = 1024
_NBLK = _B // _BLK


def _log_sigmoid(x):
    return jnp.minimum(x, 0.0) - jnp.log1p(jnp.exp(-jnp.abs(x)))


def _tc_body(cen_ref, ctx_ref, neg_ref, hcen_ref, hctx_ref, hneg_ref,
             out_ref, c_acc, s_acc):
    i = pl.program_id(0)

    @pl.when(i == 0)
    def _():
        c_acc[...] = jnp.zeros_like(c_acc)
        s_acc[0, 0] = 0.0

    cen2 = cen_ref[...]
    ctx2 = ctx_ref[...]
    neg2 = neg_ref[...]
    cen = jnp.where(hcen_ref[...] == 1, cen2[:, _D:], cen2[:, :_D])
    ctx = jnp.where(hctx_ref[...] == 1, ctx2[:, _D:], ctx2[:, :_D])
    neg = jnp.where(hneg_ref[...][:, :, None] == 1,
                    neg2[:, :, _D:], neg2[:, :, :_D])
    c_acc[...] += lax.dot_general(ctx, cen, (((0,), (0,)), ((), ())),
                                  preferred_element_type=jnp.float32)
    rest = jnp.sum(neg * cen[:, None, :], axis=-1)
    s_acc[0, 0] += jnp.sum(_log_sigmoid(-rest))

    @pl.when(i == _NBLK - 1)
    def _():
        out_ref[0, 0] = -(jnp.mean(_log_sigmoid(c_acc[...])) + s_acc[0, 0])


_tc_reduce = pl.pallas_call(
    _tc_body,
    grid=(_NBLK,),
    in_specs=[
        pl.BlockSpec((_BLK, _PD), lambda i: (i, 0)),
        pl.BlockSpec((_BLK, _PD), lambda i: (i, 0)),
        pl.BlockSpec((_BLK, _K, _PD), lambda i: (i, 0, 0)),
        pl.BlockSpec((_BLK, 1), lambda i: (i, 0)),
        pl.BlockSpec((_BLK, 1), lambda i: (i, 0)),
        pl.BlockSpec((_BLK, _K), lambda i: (i, 0)),
    ],
    out_specs=pl.BlockSpec(memory_space=pltpu.SMEM),
    out_shape=jax.ShapeDtypeStruct((1, 1), jnp.float32),
    scratch_shapes=[
        pltpu.VMEM((_D, _D), jnp.float32),
        pltpu.SMEM((1, 1), jnp.float32),
    ],
)


def kernel(central, context, neg_samples, W_central, W_context):
    cen = central.astype(jnp.int32)
    ctx = context.astype(jnp.int32)
    neg = neg_samples.astype(jnp.int32)

    def prow(v):
        return jnp.where(v >= _HV, v - _HV, v)

    def half(v):
        return (v >= _HV).astype(jnp.int32)

    cen_g = prow(cen).reshape(_NW, _CEN_CH, _CH)
    ctx_g = prow(ctx).reshape(_NW, _CEN_CH, _CH)
    neg_g = prow(neg).reshape(_NW, _NEG_CH, _CH)
    tcen = _tc_transpose(W_central.T)
    tctx = _tc_transpose(W_context.T)
    emb_cen, emb_ctx, emb_neg = _sc_gather(cen_g, ctx_g, neg_g, tcen, tctx)
    loss = _tc_reduce(emb_cen, emb_ctx, emb_neg.reshape(_B, _K, _PD),
                      half(cen).reshape(_B, 1), half(ctx).reshape(_B, 1),
                      half(neg))
    return loss.reshape(())


# speedup vs baseline: 1.5990x; 1.0879x over previous
import functools

import jax
import jax.numpy as jnp
from jax import lax
from jax.experimental import pallas as pl
from jax.experimental.pallas import tpu as pltpu
from jax.experimental.pallas import tpu_sc as plsc

_B = 16384
_D = 64
_K = 20
_VOC = 1000000
_NC = 2
_NS = 16
_NW = _NC * _NS
_CH = 128
_PD = 2 * _D

_CEN_CH = _B // _NW // _CH
_NEG_ROWS = _B * _K
_NEG_CH = _NEG_ROWS // _NW // _CH

_T = 256
_FULL = (_VOC // _T) * _T
_NRANGE = _FULL // _T
_TAIL = _VOC - _FULL


_TB = 16384
_TGRID = 31
_HV = _TB * _TGRID


def _tp_body(lo_ref, hi_ref, out_ref):
    lo_t = jnp.swapaxes(lo_ref[...], 0, 1)
    hi_t = jnp.swapaxes(hi_ref[...], 0, 1)
    out_ref[...] = jnp.concatenate([lo_t, hi_t], axis=1)


def _tc_transpose(wt):
    return pl.pallas_call(
        _tp_body,
        grid=(_TGRID,),
        in_specs=[
            pl.BlockSpec((_D, _TB), lambda i: (0, i)),
            pl.BlockSpec((_D, _TB),
                         lambda i: (0, jnp.minimum(i + _TGRID,
                                                   _VOC // _TB))),
        ],
        out_specs=pl.BlockSpec((_TB, _PD), lambda i: (i, 0)),
        out_shape=jax.ShapeDtypeStruct((_HV, _PD), jnp.float32),
    )(wt, wt)


def _sc_gather_body(cen_idx, ctx_idx, neg_idx, tcen, tctx,
                    out_cen, out_ctx, out_neg,
                    idx_cen_v, idx_ctx_v, idx_neg_v, bufs, gsem, ssem):
    wid = lax.axis_index("s") * _NC + lax.axis_index("c")
    pltpu.sync_copy(cen_idx.at[wid], idx_cen_v)
    pltpu.sync_copy(ctx_idx.at[wid], idx_ctx_v)
    pltpu.sync_copy(neg_idx.at[wid], idx_neg_v)

    def run(table, idx_v, nch, out_hbm):
        base = wid * nch * _CH

        def g_start(j, s):
            pltpu.make_async_copy(table.at[idx_v.at[j]], bufs.at[s],
                                  gsem.at[s]).start()

        def g_wait(j, s):
            pltpu.make_async_copy(table.at[idx_v.at[j]], bufs.at[s],
                                  gsem.at[s]).wait()

        def s_start(j, s):
            pltpu.make_async_copy(bufs.at[s],
                                  out_hbm.at[pl.ds(base + j * _CH, _CH)],
                                  ssem.at[s]).start()

        def s_wait(j, s):
            pltpu.make_async_copy(bufs.at[s],
                                  out_hbm.at[pl.ds(base + j * _CH, _CH)],
                                  ssem.at[s]).wait()

        nhalf = nch // 2
        g_start(0, 0)
        g_start(1, 1)

        def body(j2, carry):
            j = 2 * j2
            cur = lax.rem(j2, 2) * 2
            nxt = 2 - cur
            g_wait(j, cur)
            g_wait(j + 1, cur + 1)
            s_start(j, cur)
            s_start(j + 1, cur + 1)

            @pl.when(j2 >= 1)
            def _():
                s_wait(j - 2, nxt)
                s_wait(j - 1, nxt + 1)

            @pl.when(j2 + 1 < nhalf)
            def _():
                g_start(j + 2, nxt)
                g_start(j + 3, nxt + 1)

            return carry

        lax.fori_loop(0, nhalf, body, 0)
        last = (nhalf - 1) % 2 * 2
        s_wait(nch - 2, last)
        s_wait(nch - 1, last + 1)

    run(tcen, idx_cen_v, _CEN_CH, out_cen)
    run(tctx, idx_ctx_v, _CEN_CH, out_ctx)
    run(tctx, idx_neg_v, _NEG_CH, out_neg)


_sc_gather = pl.kernel(
    _sc_gather_body,
    out_type=(
        jax.ShapeDtypeStruct((_B, _PD), jnp.float32),
        jax.ShapeDtypeStruct((_B, _PD), jnp.float32),
        jax.ShapeDtypeStruct((_NEG_ROWS, _PD), jnp.float32),
    ),
    mesh=plsc.VectorSubcoreMesh(core_axis_name="c", subcore_axis_name="s",
                                num_cores=_NC, num_subcores=_NS),
    scratch_types=[
        pltpu.VMEM((_CEN_CH, _CH), jnp.int32),
        pltpu.VMEM((_CEN_CH, _CH), jnp.int32),
        pltpu.VMEM((_NEG_CH, _CH), jnp.int32),
        pltpu.VMEM((4, _CH, _PD), jnp.float32),
        pltpu.SemaphoreType.DMA((4,)),
        pltpu.SemaphoreType.DMA((4,)),
    ],
)


_BLK = 1024
_NBLK = _B // _BLK


def _log_sigmoid(x):
    return jnp.minimum(x, 0.0) - jnp.log1p(jnp.exp(-jnp.abs(x)))


def _tc_body(cen_ref, ctx_ref, neg_ref, hcen_ref, hctx_ref, hneg_ref,
             out_ref, c_acc, s_acc):
    i = pl.program_id(0)

    @pl.when(i == 0)
    def _():
        c_acc[...] = jnp.zeros_like(c_acc)
        s_acc[0, 0] = 0.0

    cen2 = cen_ref[...]
    ctx2 = ctx_ref[...]
    neg2 = neg_ref[...]
    cen = jnp.where(hcen_ref[...] == 1, cen2[:, _D:], cen2[:, :_D])
    ctx = jnp.where(hctx_ref[...] == 1, ctx2[:, _D:], ctx2[:, :_D])
    neg = jnp.where(hneg_ref[...][:, :, None] == 1,
                    neg2[:, :, _D:], neg2[:, :, :_D])
    c_acc[...] += lax.dot_general(ctx, cen, (((0,), (0,)), ((), ())),
                                  preferred_element_type=jnp.float32)
    rest = jnp.sum(neg * cen[:, None, :], axis=-1)
    s_acc[0, 0] += jnp.sum(_log_sigmoid(-rest))

    @pl.when(i == _NBLK - 1)
    def _():
        out_ref[0, 0] = -(jnp.mean(_log_sigmoid(c_acc[...])) + s_acc[0, 0])


_tc_reduce = pl.pallas_call(
    _tc_body,
    grid=(_NBLK,),
    in_specs=[
        pl.BlockSpec((_BLK, _PD), lambda i: (i, 0)),
        pl.BlockSpec((_BLK, _PD), lambda i: (i, 0)),
        pl.BlockSpec((_BLK, _K, _PD), lambda i: (i, 0, 0)),
        pl.BlockSpec((_BLK, 1), lambda i: (i, 0)),
        pl.BlockSpec((_BLK, 1), lambda i: (i, 0)),
        pl.BlockSpec((_BLK, _K), lambda i: (i, 0)),
    ],
    out_specs=pl.BlockSpec(memory_space=pltpu.SMEM),
    out_shape=jax.ShapeDtypeStruct((1, 1), jnp.float32),
    scratch_shapes=[
        pltpu.VMEM((_D, _D), jnp.float32),
        pltpu.SMEM((1, 1), jnp.float32),
    ],
)


def kernel(central, context, neg_samples, W_central, W_context):
    cen = central.astype(jnp.int32)
    ctx = context.astype(jnp.int32)
    neg = neg_samples.astype(jnp.int32)

    def prow(v):
        return jnp.where(v >= _HV, v - _HV, v)

    def half(v):
        return (v >= _HV).astype(jnp.int32)

    cen_g = prow(cen).reshape(_NW, _CEN_CH, _CH)
    ctx_g = prow(ctx).reshape(_NW, _CEN_CH, _CH)
    neg_g = prow(neg).reshape(_NW, _NEG_CH, _CH)
    tcen = _tc_transpose(W_central.T)
    tctx = _tc_transpose(W_context.T)
    emb_cen, emb_ctx, emb_neg = _sc_gather(cen_g, ctx_g, neg_g, tcen, tctx)
    loss = _tc_reduce(emb_cen, emb_ctx, emb_neg.reshape(_B, _K, _PD),
                      half(cen).reshape(_B, 1), half(ctx).reshape(_B, 1),
                      half(neg))
    return loss.reshape(())
